# initial kernel scaffold (unmeasured)
import jax
import jax.numpy as jnp
from jax import lax
from jax.experimental import pallas as pl
from jax.experimental.pallas import tpu as pltpu

N_DEV = 32


def kernel(x, Wq, K_ext, V_ext, Wo):
    B, Sq, D = x.shape
    _, Skv, Hq_tot, Dh = K_ext.shape
    d_loc = Wq.shape[1]
    h_loc = d_loc // Dh
    rows = Sq // N_DEV

    def body(x_ref, wq_ref, k_ref, v_ref, wo_ref, out_ref,
             ctx_ref, partial_ref, rs_ref, red_ref,
             send1, recv1, send2, recv2):
        my = lax.axis_index("i")

        xf = x_ref[...].reshape(B * Sq, D)
        qf = jnp.dot(xf, wq_ref[...], preferred_element_type=jnp.float32)

        qi = lax.broadcasted_iota(jnp.int32, (Sq, Skv), 0)
        kj = lax.broadcasted_iota(jnp.int32, (Sq, Skv), 1)
        qb = qi // 64
        kb = kj // 64
        mask = (qb == kb) | ((kb % 4) == (qb % 4))

        for b in range(B):
            for h in range(h_loc):
                hg = my * h_loc + h
                q = qf[b * Sq:(b + 1) * Sq, h * Dh:(h + 1) * Dh]
                k = pl.load(
                    k_ref, (b, slice(None), pl.ds(hg, 1), slice(None))
                ).reshape(Skv, Dh)
                v = pl.load(
                    v_ref, (b, slice(None), pl.ds(hg, 1), slice(None))
                ).reshape(Skv, Dh)
                s = lax.dot_general(
                    q, k, (((1,), (1,)), ((), ())),
                    preferred_element_type=jnp.float32,
                ) * 0.125
                s = jnp.where(mask, s, jnp.float32(-1e9))
                s = s - jnp.max(s, axis=1, keepdims=True)
                w = jnp.exp(s)
                w = w / jnp.sum(w, axis=1, keepdims=True)
                ctx = jnp.dot(w, v, preferred_element_type=jnp.float32)
                ctx_ref[b * Sq:(b + 1) * Sq, h * Dh:(h + 1) * Dh] = ctx

        pf = jnp.dot(ctx_ref[...], wo_ref[...], preferred_element_type=jnp.float32)
        partial_ref[...] = pf.reshape(B, Sq, D)

        bar = pltpu.get_barrier_semaphore()
        for off in range(1, N_DEV):
            peer = lax.rem(my + off, N_DEV)
            pl.semaphore_signal(bar, inc=1, device_id=(peer,),
                                device_id_type=pl.DeviceIdType.MESH)
        pl.semaphore_wait(bar, N_DEV - 1)

        p1 = []
        for off in range(1, N_DEV):
            dst = lax.rem(my + off, N_DEV)
            r = pltpu.make_async_remote_copy(
                src_ref=partial_ref.at[:, pl.ds(dst * rows, rows), :],
                dst_ref=rs_ref.at[off - 1],
                send_sem=send1.at[off - 1],
                recv_sem=recv1.at[off - 1],
                device_id=(dst,),
                device_id_type=pl.DeviceIdType.MESH,
            )
            r.start()
            p1.append(r)
        for r in p1:
            r.wait()

        red = pl.load(
            partial_ref, (slice(None), pl.ds(my * rows, rows), slice(None))
        )
        red = red + jnp.sum(rs_ref[...], axis=0)
        red_ref[...] = red

        p2 = []
        for off in range(1, N_DEV):
            dst = lax.rem(my + off, N_DEV)
            r = pltpu.make_async_remote_copy(
                src_ref=red_ref,
                dst_ref=out_ref.at[:, pl.ds(my * rows, rows), :],
                send_sem=send2.at[off - 1],
                recv_sem=recv2.at[off - 1],
                device_id=(dst,),
                device_id_type=pl.DeviceIdType.MESH,
            )
            r.start()
            p2.append(r)
        pl.store(
            out_ref, (slice(None), pl.ds(my * rows, rows), slice(None)), red
        )
        for r in p2:
            r.wait()

    return pl.pallas_call(
        body,
        out_shape=jax.ShapeDtypeStruct((B, Sq, D), jnp.float32),
        in_specs=[pl.BlockSpec(memory_space=pltpu.VMEM)] * 5,
        out_specs=pl.BlockSpec(memory_space=pltpu.VMEM),
        scratch_shapes=[
            pltpu.VMEM((B * Sq, d_loc), jnp.float32),
            pltpu.VMEM((B, Sq, D), jnp.float32),
            pltpu.VMEM((N_DEV - 1, B, rows, D), jnp.float32),
            pltpu.VMEM((B, rows, D), jnp.float32),
            pltpu.SemaphoreType.DMA((N_DEV - 1,)),
            pltpu.SemaphoreType.DMA((N_DEV - 1,)),
            pltpu.SemaphoreType.DMA((N_DEV - 1,)),
            pltpu.SemaphoreType.DMA((N_DEV - 1,)),
        ],
        compiler_params=pltpu.CompilerParams(collective_id=0),
    )(x, Wq, K_ext, V_ext, Wo)


# baseline (device time: 49835 ns/iter reference)
import jax
import jax.numpy as jnp
from jax import lax
from jax.experimental import pallas as pl
from jax.experimental.pallas import tpu as pltpu

N_DEV = 32


def kernel(x, Wq, K_ext, V_ext, Wo):
    B, Sq, D = x.shape
    _, Skv, Hq_tot, Dh = K_ext.shape
    d_loc = Wq.shape[1]
    h_loc = d_loc // Dh
    R = B * Sq
    rows = R // N_DEV

    K2 = K_ext.reshape(B, Skv, Hq_tot * Dh)
    V2 = V_ext.reshape(B, Skv, Hq_tot * Dh)

    def body(x_ref, wq_ref, k_ref, v_ref, wo_ref, out_ref,
             ctx_ref, partial_ref, rs_ref, red_ref,
             send1, recv1, send2, recv2):
        my = lax.axis_index("i")

        xf = x_ref[...].reshape(R, D)
        qf = jnp.dot(xf, wq_ref[...], preferred_element_type=jnp.float32)

        qi = lax.broadcasted_iota(jnp.int32, (Sq, Skv), 0)
        kj = lax.broadcasted_iota(jnp.int32, (Sq, Skv), 1)
        qb = qi // 64
        kb = kj // 64
        mask = (qb == kb) | ((kb % 4) == (qb % 4))

        for b in range(B):
            k_blk = k_ref[b, :, pl.ds(my * d_loc, d_loc)]
            v_blk = v_ref[b, :, pl.ds(my * d_loc, d_loc)]
            for h in range(h_loc):
                q = qf[b * Sq:(b + 1) * Sq, h * Dh:(h + 1) * Dh]
                k = k_blk[:, h * Dh:(h + 1) * Dh]
                v = v_blk[:, h * Dh:(h + 1) * Dh]
                s = lax.dot_general(
                    q, k, (((1,), (1,)), ((), ())),
                    preferred_element_type=jnp.float32,
                ) * 0.125
                s = jnp.where(mask, s, jnp.float32(-1e9))
                s = s - jnp.max(s, axis=1, keepdims=True)
                w = jnp.exp(s)
                w = w / jnp.sum(w, axis=1, keepdims=True)
                ctx = jnp.dot(w, v, preferred_element_type=jnp.float32)
                ctx_ref[b * Sq:(b + 1) * Sq, h * Dh:(h + 1) * Dh] = ctx

        partial_ref[...] = jnp.dot(
            ctx_ref[...], wo_ref[...], preferred_element_type=jnp.float32
        )

        bar = pltpu.get_barrier_semaphore()
        for off in range(1, N_DEV):
            peer = lax.rem(my + off, N_DEV)
            pl.semaphore_signal(bar, inc=1, device_id=(peer,),
                                device_id_type=pl.DeviceIdType.MESH)
        pl.semaphore_wait(bar, N_DEV - 1)

        p1 = []
        for off in range(1, N_DEV):
            dst = lax.rem(my + off, N_DEV)
            r = pltpu.make_async_remote_copy(
                src_ref=partial_ref.at[pl.ds(dst * rows, rows), :],
                dst_ref=rs_ref.at[off - 1],
                send_sem=send1.at[off - 1],
                recv_sem=recv1.at[off - 1],
                device_id=(dst,),
                device_id_type=pl.DeviceIdType.MESH,
            )
            r.start()
            p1.append(r)
        for r in p1:
            r.wait()

        red = partial_ref[pl.ds(my * rows, rows), :]
        red = red + jnp.sum(rs_ref[...], axis=0)
        red_ref[...] = red

        p2 = []
        for off in range(1, N_DEV):
            dst = lax.rem(my + off, N_DEV)
            r = pltpu.make_async_remote_copy(
                src_ref=red_ref,
                dst_ref=out_ref.at[pl.ds(my * rows, rows), :],
                send_sem=send2.at[off - 1],
                recv_sem=recv2.at[off - 1],
                device_id=(dst,),
                device_id_type=pl.DeviceIdType.MESH,
            )
            r.start()
            p2.append(r)
        out_ref[pl.ds(my * rows, rows), :] = red
        for r in p2:
            r.wait()

    out = pl.pallas_call(
        body,
        out_shape=jax.ShapeDtypeStruct((R, D), jnp.float32),
        in_specs=[pl.BlockSpec(memory_space=pltpu.VMEM)] * 5,
        out_specs=pl.BlockSpec(memory_space=pltpu.VMEM),
        scratch_shapes=[
            pltpu.VMEM((R, d_loc), jnp.float32),
            pltpu.VMEM((R, D), jnp.float32),
            pltpu.VMEM((N_DEV - 1, rows, D), jnp.float32),
            pltpu.VMEM((rows, D), jnp.float32),
            pltpu.SemaphoreType.DMA((N_DEV - 1,)),
            pltpu.SemaphoreType.DMA((N_DEV - 1,)),
            pltpu.SemaphoreType.DMA((N_DEV - 1,)),
            pltpu.SemaphoreType.DMA((N_DEV - 1,)),
        ],
        compiler_params=pltpu.CompilerParams(collective_id=0),
    )(x, Wq, K2, V2, Wo)
    return out.reshape(B, Sq, D)


# device time: 47008 ns/iter; 1.0601x vs baseline; 1.0601x over previous
import jax
import jax.numpy as jnp
from jax import lax
from jax.experimental import pallas as pl
from jax.experimental.pallas import tpu as pltpu

N_DEV = 32


def kernel(x, Wq, K_ext, V_ext, Wo):
    B, Sq, D = x.shape
    _, Skv, Hq_tot, Dh = K_ext.shape
    d_loc = Wq.shape[1]
    h_loc = d_loc // Dh
    R = B * Sq
    rows = R // N_DEV

    K2 = K_ext.reshape(B, Skv, Hq_tot * Dh)
    V2 = V_ext.reshape(B, Skv, Hq_tot * Dh)

    def body(x_ref, wq_ref, k_ref, v_ref, wo_ref, out_ref,
             ctx_ref, partial_ref, rs_ref, red_ref, kv_ref,
             send1, recv1, send2, recv2, kv_sem):
        my = lax.axis_index("i")

        bar = pltpu.get_barrier_semaphore()
        for off in range(1, N_DEV):
            peer = lax.rem(my + off, N_DEV)
            pl.semaphore_signal(bar, inc=1, device_id=(peer,),
                                device_id_type=pl.DeviceIdType.MESH)

        kv_dmas = []
        for i, src in enumerate((k_ref, v_ref)):
            for b in range(B):
                c = pltpu.make_async_copy(
                    src.at[b, :, pl.ds(my * d_loc, d_loc)],
                    kv_ref.at[i, b],
                    kv_sem.at[2 * i + b],
                )
                c.start()
                kv_dmas.append(c)

        xf = x_ref[...].reshape(R, D)
        qf = jnp.dot(xf, wq_ref[...], preferred_element_type=jnp.float32)

        qi = lax.broadcasted_iota(jnp.int32, (Sq, Skv), 0)
        kj = lax.broadcasted_iota(jnp.int32, (Sq, Skv), 1)
        qb = qi // 64
        kb = kj // 64
        mask = (qb == kb) | ((kb % 4) == (qb % 4))

        for c in kv_dmas:
            c.wait()
        for b in range(B):
            k_blk = kv_ref[0, b]
            v_blk = kv_ref[1, b]
            for h in range(h_loc):
                q = qf[b * Sq:(b + 1) * Sq, h * Dh:(h + 1) * Dh]
                k = k_blk[:, h * Dh:(h + 1) * Dh]
                v = v_blk[:, h * Dh:(h + 1) * Dh]
                s = lax.dot_general(
                    q, k, (((1,), (1,)), ((), ())),
                    preferred_element_type=jnp.float32,
                ) * 0.125
                s = jnp.where(mask, s, jnp.float32(-1e9))
                s = s - jnp.max(s, axis=1, keepdims=True)
                w = jnp.exp(s)
                w = w / jnp.sum(w, axis=1, keepdims=True)
                ctx = jnp.dot(w, v, preferred_element_type=jnp.float32)
                ctx_ref[b * Sq:(b + 1) * Sq, h * Dh:(h + 1) * Dh] = ctx

        partial_ref[...] = jnp.dot(
            ctx_ref[...], wo_ref[...], preferred_element_type=jnp.float32
        )

        pl.semaphore_wait(bar, N_DEV - 1)

        p1 = []
        for off in range(1, N_DEV):
            dst = lax.rem(my + off, N_DEV)
            r = pltpu.make_async_remote_copy(
                src_ref=partial_ref.at[pl.ds(dst * rows, rows), :],
                dst_ref=rs_ref.at[off - 1],
                send_sem=send1.at[off - 1],
                recv_sem=recv1.at[off - 1],
                device_id=(dst,),
                device_id_type=pl.DeviceIdType.MESH,
            )
            r.start()
            p1.append(r)
        for r in p1:
            r.wait()

        red = partial_ref[pl.ds(my * rows, rows), :]
        red = red + jnp.sum(rs_ref[...], axis=0)
        red_ref[...] = red

        p2 = []
        for off in range(1, N_DEV):
            dst = lax.rem(my + off, N_DEV)
            r = pltpu.make_async_remote_copy(
                src_ref=red_ref,
                dst_ref=out_ref.at[pl.ds(my * rows, rows), :],
                send_sem=send2.at[off - 1],
                recv_sem=recv2.at[off - 1],
                device_id=(dst,),
                device_id_type=pl.DeviceIdType.MESH,
            )
            r.start()
            p2.append(r)
        out_ref[pl.ds(my * rows, rows), :] = red
        for r in p2:
            r.wait()

    out = pl.pallas_call(
        body,
        out_shape=jax.ShapeDtypeStruct((R, D), jnp.float32),
        in_specs=[
            pl.BlockSpec(memory_space=pltpu.VMEM),
            pl.BlockSpec(memory_space=pltpu.VMEM),
            pl.BlockSpec(memory_space=pltpu.MemorySpace.HBM),
            pl.BlockSpec(memory_space=pltpu.MemorySpace.HBM),
            pl.BlockSpec(memory_space=pltpu.VMEM),
        ],
        out_specs=pl.BlockSpec(memory_space=pltpu.VMEM),
        scratch_shapes=[
            pltpu.VMEM((R, d_loc), jnp.float32),
            pltpu.VMEM((R, D), jnp.float32),
            pltpu.VMEM((N_DEV - 1, rows, D), jnp.float32),
            pltpu.VMEM((rows, D), jnp.float32),
            pltpu.VMEM((2, B, Skv, d_loc), jnp.float32),
            pltpu.SemaphoreType.DMA((N_DEV - 1,)),
            pltpu.SemaphoreType.DMA((N_DEV - 1,)),
            pltpu.SemaphoreType.DMA((N_DEV - 1,)),
            pltpu.SemaphoreType.DMA((N_DEV - 1,)),
            pltpu.SemaphoreType.DMA((4,)),
        ],
        compiler_params=pltpu.CompilerParams(collective_id=0),
    )(x, Wq, K2, V2, Wo)
    return out.reshape(B, Sq, D)
